# baseline (speedup 1.0000x reference)
import functools

import jax
import jax.numpy as jnp
from jax import lax
from jax.experimental import pallas as pl
from jax.experimental.pallas import tpu as pltpu
from jax.experimental.pallas import tpu_sc as plsc

BATCH = 16384
EMBED_DIM = 64
TILE_ROWS = 8
LANES = 16
NUM_WORKERS = 32
B_PER_W = BATCH // NUM_WORKERS
CHUNK = 16
NCHUNK = B_PER_W // CHUNK


def _sc_body(users_hbm, items_hbm, utab_hbm, itab_hbm, out_hbm,
             idx_u, idx_i, u_a, i_a, u_b, i_b, i_sh_a, i_sh_b, out_v,
             sem_a, sem_b):
    c = lax.axis_index("c")
    s = lax.axis_index("s")
    wid = s * 2 + c
    base = wid * B_PER_W

    pltpu.sync_copy(users_hbm.at[pl.ds(base, B_PER_W)], idx_u)
    pltpu.sync_copy(items_hbm.at[pl.ds(base, B_PER_W)], idx_i)

    def fire(cn, u_buf, i_sh, sem):
        def fire_g(g, carry):
            vu = idx_u[pl.ds(cn * CHUNK + g * LANES, LANES)] >> 3
            vi = idx_i[pl.ds(cn * CHUNK + g * LANES, LANES)] >> 3
            for j in range(LANES):
                ublk = pl.multiple_of(vu[j] * TILE_ROWS, TILE_ROWS)
                iblk = pl.multiple_of(vi[j] * TILE_ROWS, TILE_ROWS)
                dst = pl.ds((g * LANES + j) * TILE_ROWS, TILE_ROWS)
                pltpu.make_async_copy(
                    utab_hbm.at[pl.ds(ublk, TILE_ROWS), :], u_buf.at[dst, :],
                    sem).start()
                pltpu.make_async_copy(
                    itab_hbm.at[pl.ds(iblk, TILE_ROWS), :],
                    i_sh.at[s, dst, :], sem).start()
            return carry

        lax.fori_loop(0, CHUNK // LANES, fire_g, 0)

    def drain(u_buf, i_sh, i_buf, sem):
        def drain_j(j, carry):
            pltpu.make_async_copy(
                utab_hbm.at[pl.ds(0, TILE_ROWS), :],
                u_buf.at[pl.ds(0, TILE_ROWS), :], sem).wait()
            pltpu.make_async_copy(
                utab_hbm.at[pl.ds(0, TILE_ROWS), :],
                i_sh.at[s, pl.ds(0, TILE_ROWS), :], sem).wait()
            return carry

        lax.fori_loop(0, CHUNK, drain_j, 0)
        pltpu.sync_copy(i_sh.at[s], i_buf)

    def compute(cn, u_buf, i_buf):
        def g_body(g, carry):
            bsl = pl.ds(cn * CHUNK + g * LANES, LANES)
            rows_u = (g * LANES + lax.iota(jnp.int32, LANES)) * TILE_ROWS \
                + (idx_u[bsl] & 7)
            rows_i = (g * LANES + lax.iota(jnp.int32, LANES)) * TILE_ROWS \
                + (idx_i[bsl] & 7)
            cols = jnp.zeros((LANES,), jnp.int32)
            acc0 = jnp.zeros((LANES,), jnp.float32)
            acc1 = jnp.zeros((LANES,), jnp.float32)
            for d in range(EMBED_DIM):
                u = plsc.load_gather(u_buf, [rows_u, cols])
                v = plsc.load_gather(i_buf, [rows_i, cols])
                if d % 2 == 0:
                    acc0 = acc0 + u * v
                else:
                    acc1 = acc1 + u * v
                if d != EMBED_DIM - 1:
                    cols = cols + 1
            out_v[pl.ds(cn * CHUNK + g * LANES, LANES)] = acc0 + acc1
            return carry

        lax.fori_loop(0, CHUNK // LANES, g_body, 0)

    fire(0, u_a, i_sh_a, sem_a)

    def pair_body(p, carry):
        fire(2 * p + 1, u_b, i_sh_b, sem_b)
        drain(u_a, i_sh_a, i_a, sem_a)
        compute(2 * p, u_a, i_a)

        @pl.when(p < NCHUNK // 2 - 1)
        def _():
            fire(2 * p + 2, u_a, i_sh_a, sem_a)

        drain(u_b, i_sh_b, i_b, sem_b)
        compute(2 * p + 1, u_b, i_b)
        return carry

    lax.fori_loop(0, NCHUNK // 2, pair_body, 0)

    pltpu.sync_copy(out_v, out_hbm.at[pl.ds(base, B_PER_W)])


@functools.partial(
    pl.kernel,
    out_type=jax.ShapeDtypeStruct((BATCH,), jnp.float32),
    mesh=plsc.VectorSubcoreMesh(core_axis_name="c", subcore_axis_name="s"),
    compiler_params=pltpu.CompilerParams(needs_layout_passes=False),
    scratch_types=[
        pltpu.VMEM((B_PER_W,), jnp.int32),
        pltpu.VMEM((B_PER_W,), jnp.int32),
        pltpu.VMEM((CHUNK * TILE_ROWS, EMBED_DIM), jnp.float32),
        pltpu.VMEM((CHUNK * TILE_ROWS, EMBED_DIM), jnp.float32),
        pltpu.VMEM((CHUNK * TILE_ROWS, EMBED_DIM), jnp.float32),
        pltpu.VMEM((CHUNK * TILE_ROWS, EMBED_DIM), jnp.float32),
        pltpu.VMEM_SHARED((16, CHUNK * TILE_ROWS, EMBED_DIM), jnp.float32),
        pltpu.VMEM_SHARED((16, CHUNK * TILE_ROWS, EMBED_DIM), jnp.float32),
        pltpu.VMEM((B_PER_W,), jnp.float32),
        pltpu.SemaphoreType.DMA,
        pltpu.SemaphoreType.DMA,
    ],
)
def _sc_kernel(users, items, utab, itab, out, idx_u, idx_i, u_a, i_a, u_b,
               i_b, i_sh_a, i_sh_b, out_v, sem_a, sem_b):
    _sc_body(users, items, utab, itab, out, idx_u, idx_i, u_a, i_a, u_b, i_b,
             i_sh_a, i_sh_b, out_v, sem_a, sem_b)


def kernel(users, items, user_table, item_table):
    return _sc_kernel(users.astype(jnp.int32), items.astype(jnp.int32),
                      user_table, item_table)


# speedup vs baseline: 1.0158x; 1.0158x over previous
import functools

import jax
import jax.numpy as jnp
from jax import lax
from jax.experimental import pallas as pl
from jax.experimental.pallas import tpu as pltpu
from jax.experimental.pallas import tpu_sc as plsc

BATCH = 16384
EMBED_DIM = 64
TILE_ROWS = 8
LANES = 16
NUM_WORKERS = 32
B_PER_W = BATCH // NUM_WORKERS
CHUNK = 16
NCHUNK = B_PER_W // CHUNK


def _sc_body(users_hbm, items_hbm, utab_hbm, itab_hbm, out_hbm,
             idx_u, idx_i, u_a, i_a, u_b, i_b, out_v, sem_a, sem_b):
    c = lax.axis_index("c")
    s = lax.axis_index("s")
    wid = s * 2 + c
    base = wid * B_PER_W

    pltpu.sync_copy(users_hbm.at[pl.ds(base, B_PER_W)], idx_u)
    pltpu.sync_copy(items_hbm.at[pl.ds(base, B_PER_W)], idx_i)

    def fire(cn, u_buf, i_buf, sem):
        def fire_g(g, carry):
            vu = idx_u[pl.ds(cn * CHUNK + g * LANES, LANES)] >> 3
            vi = idx_i[pl.ds(cn * CHUNK + g * LANES, LANES)] >> 3
            for j in range(LANES):
                ublk = pl.multiple_of(vu[j] * TILE_ROWS, TILE_ROWS)
                iblk = pl.multiple_of(vi[j] * TILE_ROWS, TILE_ROWS)
                dst = pl.ds((g * LANES + j) * TILE_ROWS, TILE_ROWS)
                pltpu.make_async_copy(
                    utab_hbm.at[pl.ds(ublk, TILE_ROWS), :], u_buf.at[dst, :],
                    sem).start()
                pltpu.make_async_copy(
                    itab_hbm.at[pl.ds(iblk, TILE_ROWS), :], i_buf.at[dst, :],
                    sem).start()
            return carry

        lax.fori_loop(0, CHUNK // LANES, fire_g, 0)

    def drain(u_buf, i_buf, sem):
        def drain_j(j, carry):
            pltpu.make_async_copy(
                utab_hbm.at[pl.ds(0, TILE_ROWS), :],
                u_buf.at[pl.ds(0, TILE_ROWS), :], sem).wait()
            pltpu.make_async_copy(
                utab_hbm.at[pl.ds(0, TILE_ROWS), :],
                i_buf.at[pl.ds(0, TILE_ROWS), :], sem).wait()
            return carry

        lax.fori_loop(0, CHUNK, drain_j, 0)

    def compute(cn, u_buf, i_buf):
        def g_body(g, carry):
            bsl = pl.ds(cn * CHUNK + g * LANES, LANES)
            rows_u = (g * LANES + lax.iota(jnp.int32, LANES)) * TILE_ROWS \
                + (idx_u[bsl] & 7)
            rows_i = (g * LANES + lax.iota(jnp.int32, LANES)) * TILE_ROWS \
                + (idx_i[bsl] & 7)
            cols = jnp.zeros((LANES,), jnp.int32)
            acc0 = jnp.zeros((LANES,), jnp.float32)
            acc1 = jnp.zeros((LANES,), jnp.float32)
            for d in range(EMBED_DIM):
                u = plsc.load_gather(u_buf, [rows_u, cols])
                v = plsc.load_gather(i_buf, [rows_i, cols])
                if d % 2 == 0:
                    acc0 = acc0 + u * v
                else:
                    acc1 = acc1 + u * v
                if d != EMBED_DIM - 1:
                    cols = cols + 1
            out_v[pl.ds(cn * CHUNK + g * LANES, LANES)] = acc0 + acc1
            return carry

        lax.fori_loop(0, CHUNK // LANES, g_body, 0)

    fire(0, u_a, i_a, sem_a)

    def pair_body(p, carry):
        fire(2 * p + 1, u_b, i_b, sem_b)
        drain(u_a, i_a, sem_a)
        compute(2 * p, u_a, i_a)

        @pl.when(p < NCHUNK // 2 - 1)
        def _():
            fire(2 * p + 2, u_a, i_a, sem_a)

        drain(u_b, i_b, sem_b)
        compute(2 * p + 1, u_b, i_b)
        return carry

    lax.fori_loop(0, NCHUNK // 2, pair_body, 0)

    pltpu.sync_copy(out_v, out_hbm.at[pl.ds(base, B_PER_W)])


@functools.partial(
    pl.kernel,
    out_type=jax.ShapeDtypeStruct((BATCH,), jnp.float32),
    mesh=plsc.VectorSubcoreMesh(core_axis_name="c", subcore_axis_name="s"),
    compiler_params=pltpu.CompilerParams(needs_layout_passes=False),
    scratch_types=[
        pltpu.VMEM((B_PER_W,), jnp.int32),
        pltpu.VMEM((B_PER_W,), jnp.int32),
        pltpu.VMEM((CHUNK * TILE_ROWS, EMBED_DIM), jnp.float32),
        pltpu.VMEM((CHUNK * TILE_ROWS, EMBED_DIM), jnp.float32),
        pltpu.VMEM((CHUNK * TILE_ROWS, EMBED_DIM), jnp.float32),
        pltpu.VMEM((CHUNK * TILE_ROWS, EMBED_DIM), jnp.float32),
        pltpu.VMEM((B_PER_W,), jnp.float32),
        pltpu.SemaphoreType.DMA,
        pltpu.SemaphoreType.DMA,
    ],
)
def _sc_kernel(users, items, utab, itab, out, idx_u, idx_i, u_a, i_a, u_b,
               i_b, out_v, sem_a, sem_b):
    _sc_body(users, items, utab, itab, out, idx_u, idx_i, u_a, i_a, u_b, i_b,
             out_v, sem_a, sem_b)


def kernel(users, items, user_table, item_table):
    return _sc_kernel(users.astype(jnp.int32), items.astype(jnp.int32),
                      user_table, item_table)
